# Initial kernel scaffold; baseline (speedup 1.0000x reference)
import jax
import jax.numpy as jnp
from jax.experimental import pallas as pl


def kernel(raw, t, src, tar, n_mask, time_w, time_b, Wz, Uz, bz, Wr, Ur, br, Wh, Uh, bh, W1, W2, b_emb, Wl, bl):
    raise NotImplementedError("write your pallas kernel here")


# speedup vs baseline: 23.8768x; 23.8768x over previous
import math

import jax
import jax.numpy as jnp
from jax.experimental import pallas as pl

_B = 16
_N = 10000
_LATENT = 128
_NJ = 9

_CE = [(-1.0) ** j / math.factorial(2 * j) for j in range(_NJ)]
_CO = [(-1.0) ** j / math.factorial(2 * j + 1) for j in range(_NJ)]

_NT = (((1,), (1,)), ((), ()))


def _tgn_body(t_ref, raw_ref, m_ref, src_ref, tar_ref, srcr_ref, tarr_ref,
              w_ref, tb_ref,
              wz_r_ref, wz_m_ref, wz_d_ref,
              wr_r_ref, wr_m_ref, wr_d_ref,
              wh_r_ref, wh_m_ref, wh_d_ref,
              uz_ref, ur_ref, uh_ref,
              bz_ref, br_ref, bh_ref,
              w1_r_ref, w1_m_ref, w1_d_ref,
              w2_r_ref, w2_m_ref, w2_d_ref,
              bemb_ref, wl_ref, bl_ref, out_ref):
    f32 = jnp.float32
    t = t_ref[...]
    m = m_ref[...]
    raw = raw_ref[...]
    src = src_ref[...]
    tar = tar_ref[...]
    srcr = srcr_ref[...]
    tarr = tarr_ref[...]
    w = w_ref[...]
    tb = tb_ref[...]

    li = jax.lax.broadcasted_iota(jnp.int32, (_B, _N), 1)
    oh_src = (li == src).astype(f32)
    oh_tar = (li == tar).astype(f32)

    def nt(a, b):
        return jax.lax.dot_general(a, b, _NT, preferred_element_type=f32)

    g_t_src = nt(t, oh_src)
    g_t_tar = nt(t, oh_tar)
    g_raw_src = nt(raw, oh_src)
    g_raw_tar = nt(raw, oh_tar)
    g_m_src = nt(m, oh_src)
    g_m_tar = nt(m, oh_tar)

    ri16 = jax.lax.broadcasted_iota(jnp.int32, (_B, _B), 0)
    ci16 = jax.lax.broadcasted_iota(jnp.int32, (_B, _B), 1)
    eye = (ri16 == ci16).astype(f32)

    def diag(g):
        return jnp.sum(g * eye, axis=1, keepdims=True)

    t_src = diag(g_t_src)
    t_tar = diag(g_t_tar)
    raw_src = diag(g_raw_src)
    raw_tar = diag(g_raw_tar)
    m_tar_d = diag(g_m_tar)

    dt_src = jnp.cos(t_src * w + tb)
    dt_tar = jnp.cos(t_tar * w + tb)

    z_s = jax.nn.sigmoid(raw_src * wz_r_ref[...] + jnp.dot(dt_src, wz_d_ref[...], preferred_element_type=f32) + bz_ref[...])
    n_s = jnp.tanh(raw_src * wh_r_ref[...] + jnp.dot(dt_src, wh_d_ref[...], preferred_element_type=f32) + bh_ref[...])
    new_src = z_s * n_s

    eq_ts = tar == srcr
    rank_ts = jnp.max(jnp.where(eq_ts, ci16 + 1, 0), axis=1, keepdims=True)
    sel_ts = ((ci16 + 1) == rank_ts).astype(f32)
    mem_tar = jnp.dot(sel_ts, new_src, preferred_element_type=f32)

    wz_m = wz_m_ref[...]
    z_t = jax.nn.sigmoid(raw_tar * wz_r_ref[...]
                         + jnp.dot(mem_tar, wz_m, preferred_element_type=f32)
                         + jnp.dot(dt_tar, wz_d_ref[...], preferred_element_type=f32)
                         + jnp.dot(mem_tar, uz_ref[...], preferred_element_type=f32)
                         + bz_ref[...])
    r_t = jax.nn.sigmoid(raw_tar * wr_r_ref[...]
                         + jnp.dot(mem_tar, wr_m_ref[...], preferred_element_type=f32)
                         + jnp.dot(dt_tar, wr_d_ref[...], preferred_element_type=f32)
                         + jnp.dot(mem_tar, ur_ref[...], preferred_element_type=f32)
                         + br_ref[...])
    n_t = jnp.tanh(raw_tar * wh_r_ref[...]
                   + jnp.dot(mem_tar, wh_m_ref[...], preferred_element_type=f32)
                   + jnp.dot(dt_tar, wh_d_ref[...], preferred_element_type=f32)
                   + jnp.dot(r_t * mem_tar, uh_ref[...], preferred_element_type=f32)
                   + bh_ref[...])
    new_tar = (1.0 - z_t) * mem_tar + z_t * n_t

    eq_tt = tar == tarr
    rank_tt = jnp.max(jnp.where(eq_tt, ci16 + 1, 0), axis=1, keepdims=True)
    sel_tt = ((ci16 + 1) == rank_tt).astype(f32)
    tar_hid = jnp.dot(sel_tt, new_tar, preferred_element_type=f32)

    li16 = jax.lax.broadcasted_iota(jnp.int32, (1, _B), 1)
    d_ss = src == srcr
    lastw_ss = jnp.max(jnp.where(d_ss, ri16 + 1, 0), axis=0, keepdims=True)
    in_tar = jnp.max(jnp.where(tar == srcr, 1, 0), axis=0, keepdims=True)
    surv_src = ((lastw_ss == li16 + 1) & (in_tar == 0)).astype(f32)
    lastw_tt = jnp.max(jnp.where(eq_tt, ri16 + 1, 0), axis=0, keepdims=True)
    surv_tar = (lastw_tt == li16 + 1).astype(f32)

    a1 = g_m_src * (1.0 - eq_ts.astype(f32)) * surv_src
    a2 = g_m_tar * (1.0 - eq_tt.astype(f32)) * surv_tar
    agg_hid = (jnp.dot(a1, new_src, preferred_element_type=f32)
               + jnp.dot(a2, new_tar, preferred_element_type=f32))

    s = m
    psums = [jnp.sum(s, axis=1, keepdims=True)]
    for _ in range(1, 2 * _NJ):
        s = s * t
        psums.append(jnp.sum(s, axis=1, keepdims=True))
    pe = jnp.concatenate([psums[2 * j] * _CE[j] for j in range(_NJ)], axis=1)
    po = jnp.concatenate([psums[2 * j + 1] * _CO[j] for j in range(_NJ)], axis=1)
    w2 = w * w
    we_rows = [jnp.ones_like(w)]
    wo_rows = [w]
    for _ in range(1, _NJ):
        we_rows.append(we_rows[-1] * w2)
        wo_rows.append(wo_rows[-1] * w2)
    we = jnp.concatenate(we_rows, axis=0)
    wo = jnp.concatenate(wo_rows, axis=0)
    ecos = jnp.dot(pe, we, preferred_element_type=f32)
    esin = jnp.dot(po, wo, preferred_element_type=f32)
    ctb = jnp.cos(tb)
    stb = jnp.sin(tb)
    agg_enc = ctb * ecos - stb * esin - m_tar_d * dt_tar
    agg_raw = jnp.sum(m * raw, axis=1, keepdims=True) - m_tar_d * raw_tar

    pre = (raw_tar * w1_r_ref[...]
           + jnp.dot(tar_hid, w1_m_ref[...], preferred_element_type=f32)
           + jnp.dot(ctb, w1_d_ref[...], preferred_element_type=f32)
           + agg_raw * w2_r_ref[...]
           + jnp.dot(agg_hid, w2_m_ref[...], preferred_element_type=f32)
           + jnp.dot(agg_enc, w2_d_ref[...], preferred_element_type=f32)
           + bemb_ref[...])
    z = jax.nn.relu(pre)
    out_ref[...] = jnp.dot(z, wl_ref[...], preferred_element_type=f32) + bl_ref[...]


def kernel(raw, t, src, tar, n_mask, time_w, time_b, Wz, Uz, bz, Wr, Ur, br,
           Wh, Uh, bh, W1, W2, b_emb, Wl, bl):
    t2 = t[:, :, 0]
    raw2 = raw[:, :, 0]
    srcr = jnp.reshape(src, (1, _B))
    tarr = jnp.reshape(tar, (1, _B))
    tb = jnp.reshape(time_b, (1, _LATENT))

    def rows(wm):
        return wm[0:1], wm[1:1 + _LATENT], wm[1 + _LATENT:1 + 2 * _LATENT]

    wz_r, wz_m, wz_d = rows(Wz)
    wr_r, wr_m, wr_d = rows(Wr)
    wh_r, wh_m, wh_d = rows(Wh)
    w1_r, w1_m, w1_d = rows(W1)
    w2_r, w2_m, w2_d = rows(W2)

    return pl.pallas_call(
        _tgn_body,
        out_shape=jax.ShapeDtypeStruct((_B, 1), jnp.float32),
    )(t2, raw2, n_mask, src, tar, srcr, tarr,
      time_w, tb,
      wz_r, wz_m, wz_d, wr_r, wr_m, wr_d, wh_r, wh_m, wh_d,
      Uz, Ur, Uh,
      jnp.reshape(bz, (1, _LATENT)), jnp.reshape(br, (1, _LATENT)),
      jnp.reshape(bh, (1, _LATENT)),
      w1_r, w1_m, w1_d, w2_r, w2_m, w2_d,
      jnp.reshape(b_emb, (1, _LATENT)), Wl, jnp.reshape(bl, (1, 1)))


# speedup vs baseline: 38.9877x; 1.6329x over previous
import math

import jax
import jax.numpy as jnp
from jax.experimental import pallas as pl

_B = 16
_N = 10000
_LATENT = 128
_NJ = 9

_CE = [(-1.0) ** j / math.factorial(2 * j) for j in range(_NJ)]
_CO = [(-1.0) ** j / math.factorial(2 * j + 1) for j in range(_NJ)]

_NT = (((1,), (1,)), ((), ()))


def _tgn_body(t_ref, raw_ref, m_ref, stc_ref, str_ref, w_ref,
              wz_ref, wr_ref, wh_ref, uz_ref, ur_ref, uh_ref,
              w1_ref, w2_ref, wl_ref, bias_ref, out_ref):
    f32 = jnp.float32
    t = t_ref[...]
    m = m_ref[...]
    raw = raw_ref[...]
    stc = stc_ref[...]
    strow = str_ref[...]
    src = stc[:, 0:1]
    tar = stc[:, 1:2]
    srcr = strow[0:1]
    tarr = strow[1:2]
    w = w_ref[...]
    bias = bias_ref[...]
    bz = bias[0:1]
    br = bias[1:2]
    bh = bias[2:3]
    bemb = bias[3:4]
    tb = bias[4:5]
    bl = bias[5:6, 0:1]
    wz_r, wz_m, wz_d = wz_ref[0:1], wz_ref[1:1 + _LATENT], wz_ref[1 + _LATENT:]
    wr_r, wr_m, wr_d = wr_ref[0:1], wr_ref[1:1 + _LATENT], wr_ref[1 + _LATENT:]
    wh_r, wh_m, wh_d = wh_ref[0:1], wh_ref[1:1 + _LATENT], wh_ref[1 + _LATENT:]
    w1_r, w1_m, w1_d = w1_ref[0:1], w1_ref[1:1 + _LATENT], w1_ref[1 + _LATENT:]
    w2_r, w2_m, w2_d = w2_ref[0:1], w2_ref[1:1 + _LATENT], w2_ref[1 + _LATENT:]

    li = jax.lax.broadcasted_iota(jnp.int32, (_B, _N), 1)
    oh_src = (li == src).astype(f32)
    oh_tar = (li == tar).astype(f32)

    def nt(a, b):
        return jax.lax.dot_general(a, b, _NT, preferred_element_type=f32)

    g_t_src = nt(t, oh_src)
    g_t_tar = nt(t, oh_tar)
    g_raw_src = nt(raw, oh_src)
    g_raw_tar = nt(raw, oh_tar)
    g_m_src = nt(m, oh_src)
    g_m_tar = nt(m, oh_tar)

    ri16 = jax.lax.broadcasted_iota(jnp.int32, (_B, _B), 0)
    ci16 = jax.lax.broadcasted_iota(jnp.int32, (_B, _B), 1)
    eye = (ri16 == ci16).astype(f32)

    def diag(g):
        return jnp.sum(g * eye, axis=1, keepdims=True)

    t_src = diag(g_t_src)
    t_tar = diag(g_t_tar)
    raw_src = diag(g_raw_src)
    raw_tar = diag(g_raw_tar)
    m_tar_d = diag(g_m_tar)

    dt_src = jnp.cos(t_src * w + tb)
    dt_tar = jnp.cos(t_tar * w + tb)

    z_s = jax.nn.sigmoid(raw_src * wz_r + jnp.dot(dt_src, wz_d, preferred_element_type=f32) + bz)
    n_s = jnp.tanh(raw_src * wh_r + jnp.dot(dt_src, wh_d, preferred_element_type=f32) + bh)
    new_src = z_s * n_s

    eq_ts = tar == srcr
    rank_ts = jnp.max(jnp.where(eq_ts, ci16 + 1, 0), axis=1, keepdims=True)
    sel_ts = ((ci16 + 1) == rank_ts).astype(f32)
    mem_tar = jnp.dot(sel_ts, new_src, preferred_element_type=f32)

    z_t = jax.nn.sigmoid(raw_tar * wz_r
                         + jnp.dot(mem_tar, wz_m, preferred_element_type=f32)
                         + jnp.dot(dt_tar, wz_d, preferred_element_type=f32)
                         + jnp.dot(mem_tar, uz_ref[...], preferred_element_type=f32)
                         + bz)
    r_t = jax.nn.sigmoid(raw_tar * wr_r
                         + jnp.dot(mem_tar, wr_m, preferred_element_type=f32)
                         + jnp.dot(dt_tar, wr_d, preferred_element_type=f32)
                         + jnp.dot(mem_tar, ur_ref[...], preferred_element_type=f32)
                         + br)
    n_t = jnp.tanh(raw_tar * wh_r
                   + jnp.dot(mem_tar, wh_m, preferred_element_type=f32)
                   + jnp.dot(dt_tar, wh_d, preferred_element_type=f32)
                   + jnp.dot(r_t * mem_tar, uh_ref[...], preferred_element_type=f32)
                   + bh)
    new_tar = (1.0 - z_t) * mem_tar + z_t * n_t

    eq_tt = tar == tarr
    rank_tt = jnp.max(jnp.where(eq_tt, ci16 + 1, 0), axis=1, keepdims=True)
    sel_tt = ((ci16 + 1) == rank_tt).astype(f32)
    tar_hid = jnp.dot(sel_tt, new_tar, preferred_element_type=f32)

    li16 = jax.lax.broadcasted_iota(jnp.int32, (1, _B), 1)
    d_ss = src == srcr
    lastw_ss = jnp.max(jnp.where(d_ss, ri16 + 1, 0), axis=0, keepdims=True)
    in_tar = jnp.max(jnp.where(tar == srcr, 1, 0), axis=0, keepdims=True)
    surv_src = ((lastw_ss == li16 + 1) & (in_tar == 0)).astype(f32)
    lastw_tt = jnp.max(jnp.where(eq_tt, ri16 + 1, 0), axis=0, keepdims=True)
    surv_tar = (lastw_tt == li16 + 1).astype(f32)

    a1 = g_m_src * (1.0 - eq_ts.astype(f32)) * surv_src
    a2 = g_m_tar * (1.0 - eq_tt.astype(f32)) * surv_tar
    agg_hid = (jnp.dot(a1, new_src, preferred_element_type=f32)
               + jnp.dot(a2, new_tar, preferred_element_type=f32))

    s = m
    psums = [jnp.sum(s, axis=1, keepdims=True)]
    for _ in range(1, 2 * _NJ):
        s = s * t
        psums.append(jnp.sum(s, axis=1, keepdims=True))
    pe = jnp.concatenate([psums[2 * j] * _CE[j] for j in range(_NJ)], axis=1)
    po = jnp.concatenate([psums[2 * j + 1] * _CO[j] for j in range(_NJ)], axis=1)
    w2 = w * w
    we_rows = [jnp.ones_like(w)]
    wo_rows = [w]
    for _ in range(1, _NJ):
        we_rows.append(we_rows[-1] * w2)
        wo_rows.append(wo_rows[-1] * w2)
    we = jnp.concatenate(we_rows, axis=0)
    wo = jnp.concatenate(wo_rows, axis=0)
    ecos = jnp.dot(pe, we, preferred_element_type=f32)
    esin = jnp.dot(po, wo, preferred_element_type=f32)
    ctb = jnp.cos(tb)
    stb = jnp.sin(tb)
    agg_enc = ctb * ecos - stb * esin - m_tar_d * dt_tar
    agg_raw = jnp.sum(m * raw, axis=1, keepdims=True) - m_tar_d * raw_tar

    pre = (raw_tar * w1_r
           + jnp.dot(tar_hid, w1_m, preferred_element_type=f32)
           + jnp.dot(ctb, w1_d, preferred_element_type=f32)
           + agg_raw * w2_r
           + jnp.dot(agg_hid, w2_m, preferred_element_type=f32)
           + jnp.dot(agg_enc, w2_d, preferred_element_type=f32)
           + bemb)
    z = jax.nn.relu(pre)
    out_ref[...] = jnp.dot(z, wl_ref[...], preferred_element_type=f32) + bl


def kernel(raw, t, src, tar, n_mask, time_w, time_b, Wz, Uz, bz, Wr, Ur, br,
           Wh, Uh, bh, W1, W2, b_emb, Wl, bl):
    t2 = jnp.reshape(t, (_B, _N))
    raw2 = jnp.reshape(raw, (_B, _N))
    stc = jnp.concatenate([src, tar], axis=1)
    strow = jnp.concatenate([jnp.reshape(src, (1, _B)),
                             jnp.reshape(tar, (1, _B))], axis=0)
    bias = jnp.stack([bz, br, bh, b_emb, time_b,
                      jnp.concatenate([bl, jnp.zeros((_LATENT - 1,), jnp.float32)])])

    return pl.pallas_call(
        _tgn_body,
        out_shape=jax.ShapeDtypeStruct((_B, 1), jnp.float32),
    )(t2, raw2, n_mask, stc, strow, time_w,
      Wz, Wr, Wh, Uz, Ur, Uh, W1, W2, Wl, bias)
